# baseline (speedup 1.0000x reference)
import functools

import jax
import jax.numpy as jnp
from jax import lax
from jax.experimental import pallas as pl
from jax.experimental.pallas import tpu as pltpu
from jax.experimental.pallas import tpu_sc as plsc

N_LEVELS = 8
D_MODEL = 2048

try:
    _info = plsc.get_sparse_core_info()
    _NC, _NS = _info.num_cores, _info.num_subcores
except Exception:
    _NC, _NS = 2, 16
_NW = _NC * _NS


@functools.lru_cache(maxsize=None)
def _build(B: int, D: int, V: int):
    b_per_w = B // _NW
    CH = 16
    NBUF = 2
    n_chunks = b_per_w // CH
    n_groups = n_chunks // NBUF
    mesh = plsc.VectorSubcoreMesh(core_axis_name="c", subcore_axis_name="s")

    @functools.partial(
        pl.kernel,
        mesh=mesh,
        compiler_params=pltpu.CompilerParams(needs_layout_passes=False),
        out_type=jax.ShapeDtypeStruct((B, 1, D), jnp.float32),
        scratch_types=[
            pltpu.VMEM((b_per_w,), jnp.int32),
            pltpu.VMEM((V * D,), jnp.float32),
            [pltpu.VMEM((CH, D), jnp.float32) for _ in range(NBUF)],
            [pltpu.SemaphoreType.DMA for _ in range(NBUF)],
        ],
    )
    def lookup_kernel(idx_hbm, rep_hbm, out_hbm, idx_v, table_v, bufs, wsems):
        wid = lax.axis_index("s") * _NC + lax.axis_index("c")
        base = wid * b_per_w
        pltpu.sync_copy(idx_hbm.at[pl.ds(base, b_per_w)], idx_v)
        pltpu.sync_copy(rep_hbm.at[pl.ds(wid * V * D, V * D)], table_v)
        lanes = lax.iota(jnp.int32, 16)

        def group_body(g, carry):
            for b in range(NBUF):
                chunk = g * NBUF + b
                @pl.when(g > 0)
                def _():
                    pltpu.make_async_copy(
                        bufs[b], out_hbm.at[pl.ds(base, CH), 0], wsems[b]
                    ).wait()
                rows = idx_v[pl.ds(chunk * CH, CH)]

                @plsc.parallel_loop(0, CH, 1, unroll=4)
                def _(i, b=b, rows=rows):
                    rsplat = rows.at[jnp.full((16,), 0, jnp.int32) + i].get(
                        mode="promise_in_bounds")
                    rl = rsplat * D + lanes
                    for k in range(D // 16):
                        val = plsc.load_gather(table_v, [rl + (k * 16)])
                        bufs[b][i, pl.ds(k * 16, 16)] = val
                pltpu.async_copy(
                    bufs[b],
                    out_hbm.at[pl.ds(base + chunk * CH, CH), 0],
                    wsems[b])
            return carry

        lax.fori_loop(0, n_groups, group_body, 0, unroll=False)
        for b in range(NBUF):
            pltpu.make_async_copy(
                bufs[b], out_hbm.at[pl.ds(base, CH), 0], wsems[b]).wait()

    return lookup_kernel


def kernel(level_idx, prefix_emb):
    B = level_idx.shape[0]
    V, D = prefix_emb.shape
    rep = jnp.broadcast_to(prefix_emb[None], (_NW, V, D)).reshape(_NW * V * D)
    return _build(B, D, V)(level_idx, rep)


# speedup vs baseline: 1.2857x; 1.2857x over previous
import functools

import jax
import jax.numpy as jnp
from jax import lax
from jax.experimental import pallas as pl
from jax.experimental.pallas import tpu as pltpu
from jax.experimental.pallas import tpu_sc as plsc

N_LEVELS = 8
D_MODEL = 2048

try:
    _info = plsc.get_sparse_core_info()
    _NC, _NS = _info.num_cores, _info.num_subcores
except Exception:
    _NC, _NS = 2, 16
_NW = _NC * _NS


@functools.lru_cache(maxsize=None)
def _build(B: int, D: int, V: int):
    b_per_w = B // _NW
    CH = 16
    NBUF = 2
    n_chunks = b_per_w // CH
    n_groups = n_chunks // NBUF
    mesh = plsc.VectorSubcoreMesh(core_axis_name="c", subcore_axis_name="s")

    @functools.partial(
        pl.kernel,
        mesh=mesh,
        compiler_params=pltpu.CompilerParams(needs_layout_passes=False),
        out_type=jax.ShapeDtypeStruct((B, 1, D), jnp.float32),
        scratch_types=[
            pltpu.VMEM((b_per_w,), jnp.int32),
            pltpu.VMEM((V * D,), jnp.float32),
            [pltpu.VMEM((CH, 1, D), jnp.float32) for _ in range(NBUF)],
            [pltpu.SemaphoreType.DMA for _ in range(NBUF)],
        ],
    )
    def lookup_kernel(idx_hbm, rep_hbm, out_hbm, idx_v, table_v, bufs, wsems):
        wid = lax.axis_index("s") * _NC + lax.axis_index("c")
        base = wid * b_per_w
        pltpu.sync_copy(idx_hbm.at[pl.ds(base, b_per_w)], idx_v)
        pltpu.sync_copy(rep_hbm.at[pl.ds(wid * V * D, V * D)], table_v)
        lanes = lax.iota(jnp.int32, 16)

        def group_body(g, carry):
            for b in range(NBUF):
                chunk = g * NBUF + b
                @pl.when(g > 0)
                def _():
                    pltpu.make_async_copy(
                        bufs[b], out_hbm.at[pl.ds(base, CH)], wsems[b]
                    ).wait()
                rows = idx_v[pl.ds(chunk * CH, CH)]

                @plsc.parallel_loop(0, CH, 1, unroll=2)
                def _(i, b=b, rows=rows):
                    rsplat = rows.at[jnp.full((16,), 0, jnp.int32) + i].get(
                        mode="promise_in_bounds")
                    rl = rsplat * D + lanes
                    for k in range(D // 16):
                        val = plsc.load_gather(table_v, [rl + (k * 16)])
                        bufs[b][i, 0, pl.ds(k * 16, 16)] = val
                pltpu.async_copy(
                    bufs[b],
                    out_hbm.at[pl.ds(base + chunk * CH, CH)],
                    wsems[b])
            return carry

        lax.fori_loop(0, n_groups, group_body, 0, unroll=False)
        for b in range(NBUF):
            pltpu.make_async_copy(
                bufs[b], out_hbm.at[pl.ds(base, CH)], wsems[b]).wait()

    return lookup_kernel


def kernel(level_idx, prefix_emb):
    B = level_idx.shape[0]
    V, D = prefix_emb.shape
    rep = jnp.broadcast_to(prefix_emb[None], (_NW, V, D)).reshape(_NW * V * D)
    return _build(B, D, V)(level_idx, rep)


# speedup vs baseline: 1.6991x; 1.3216x over previous
import functools

import jax
import jax.numpy as jnp
from jax import lax
from jax.experimental import pallas as pl
from jax.experimental.pallas import tpu as pltpu
from jax.experimental.pallas import tpu_sc as plsc

N_LEVELS = 8
D_MODEL = 2048

try:
    _info = plsc.get_sparse_core_info()
    _NC, _NS = _info.num_cores, _info.num_subcores
except Exception:
    _NC, _NS = 2, 16
_NW = _NC * _NS


@functools.lru_cache(maxsize=None)
def _build(B: int, D: int, V: int):
    b_per_w = B // _NW
    CH = 16
    n_chunks = b_per_w // CH
    n_periods = n_chunks // 2
    SG = 2
    mesh = plsc.VectorSubcoreMesh(core_axis_name="c", subcore_axis_name="s")

    @functools.partial(
        pl.kernel,
        mesh=mesh,
        compiler_params=pltpu.CompilerParams(needs_layout_passes=False),
        out_type=jax.ShapeDtypeStruct((B, 1, D), jnp.float32),
        scratch_types=[
            pltpu.VMEM((b_per_w,), jnp.int32),
            pltpu.VMEM((V * D,), jnp.float32),
            [pltpu.VMEM((CH, D), jnp.float32) for _ in range(2)],
            pltpu.VMEM((CH, 1, D), jnp.float32),
            [pltpu.SemaphoreType.DMA for _ in range(2)],
            [pltpu.SemaphoreType.DMA for _ in range(3)],
        ],
    )
    def lookup_kernel(idx_hbm, rep_hbm, repf_hbm, out_hbm,
                      idx_v, table_v, sbufs, vbuf, gsems, wsems):
        wid = lax.axis_index("s") * _NC + lax.axis_index("c")
        base = wid * b_per_w
        pltpu.sync_copy(idx_hbm.at[pl.ds(base, b_per_w)], idx_v)
        pltpu.sync_copy(repf_hbm.at[pl.ds(wid * V * D, V * D)], table_v)
        lanes = lax.iota(jnp.int32, 16)
        row_off = wid * V

        def do_period(p, s):
            @pl.when(p >= 2)
            def _():
                pltpu.make_async_copy(
                    sbufs[s], out_hbm.at[pl.ds(base, CH), 0],
                    wsems[s]).wait()
            iv = idx_v[pl.ds((2 * p) * CH, CH)] + row_off
            pltpu.async_copy(rep_hbm.at[iv], sbufs[s], gsems[s])

            @pl.when(p >= 1)
            def _():
                pltpu.make_async_copy(
                    vbuf, out_hbm.at[pl.ds(base, CH)], wsems[2]).wait()
            rows = idx_v[pl.ds((2 * p + 1) * CH, CH)]

            @plsc.parallel_loop(0, CH, 1, unroll=2)
            def _(i):
                rsplat = rows.at[jnp.full((16,), 0, jnp.int32) + i].get(
                    mode="promise_in_bounds")
                rl = rsplat * D + lanes
                for k in range(D // 16):
                    val = plsc.load_gather(table_v, [rl + (k * 16)])
                    vbuf[i, 0, pl.ds(k * 16, 16)] = val

            pltpu.async_copy(
                vbuf, out_hbm.at[pl.ds(base + (2 * p + 1) * CH, CH)],
                wsems[2])
            pltpu.make_async_copy(
                rep_hbm.at[pl.ds(0, CH)], sbufs[s], gsems[s]).wait()
            pltpu.async_copy(
                sbufs[s], out_hbm.at[pl.ds(base + (2 * p) * CH, CH), 0],
                wsems[s])

        def group_body(g, carry):
            for q in range(SG):
                do_period(g * SG + q, q)
            return carry

        lax.fori_loop(0, n_periods // SG, group_body, 0, unroll=False)
        for s in range(2):
            pltpu.make_async_copy(
                sbufs[s], out_hbm.at[pl.ds(base, CH), 0], wsems[s]).wait()
        pltpu.make_async_copy(
            vbuf, out_hbm.at[pl.ds(base, CH)], wsems[2]).wait()

    return lookup_kernel


def kernel(level_idx, prefix_emb):
    B = level_idx.shape[0]
    V, D = prefix_emb.shape
    rep = jnp.broadcast_to(prefix_emb[None], (_NW, V, D))
    return _build(B, D, V)(
        level_idx, rep.reshape(_NW * V, D), rep.reshape(_NW * V * D))


# speedup vs baseline: 2.0960x; 1.2335x over previous
import functools

import jax
import jax.numpy as jnp
from jax import lax
from jax.experimental import pallas as pl
from jax.experimental.pallas import tpu as pltpu
from jax.experimental.pallas import tpu_sc as plsc

N_LEVELS = 8
D_MODEL = 2048

try:
    _info = plsc.get_sparse_core_info()
    _NC, _NS = _info.num_cores, _info.num_subcores
except Exception:
    _NC, _NS = 2, 16
_NW = _NC * _NS


@functools.lru_cache(maxsize=None)
def _build(B: int, D: int, V: int):
    b_per_w = B // _NW
    CH = 16
    n_chunks = b_per_w // CH
    n_periods = n_chunks // 2
    SG = 2
    mesh = plsc.VectorSubcoreMesh(core_axis_name="c", subcore_axis_name="s")

    @functools.partial(
        pl.kernel,
        mesh=mesh,
        compiler_params=pltpu.CompilerParams(needs_layout_passes=False),
        out_type=jax.ShapeDtypeStruct((B, 1, D), jnp.float32),
        scratch_types=[
            pltpu.VMEM((b_per_w,), jnp.int32),
            pltpu.VMEM((V * D,), jnp.float32),
            [pltpu.VMEM((CH, D), jnp.float32) for _ in range(2)],
            pltpu.VMEM((CH, 1, D), jnp.float32),
            [pltpu.SemaphoreType.DMA for _ in range(2)],
            [pltpu.SemaphoreType.DMA for _ in range(3)],
        ],
    )
    def lookup_kernel(idx_hbm, rep_hbm, repf_hbm, out_hbm,
                      idx_v, table_v, sbufs, vbuf, gsems, wsems):
        wid = lax.axis_index("s") * _NC + lax.axis_index("c")
        base = wid * b_per_w
        pltpu.sync_copy(idx_hbm.at[pl.ds(base, b_per_w)], idx_v)
        pltpu.sync_copy(repf_hbm.at[pl.ds(wid * V * D, V * D)], table_v)
        lanes = lax.iota(jnp.int32, 16)
        row_off = wid * V

        def start_stream(chunk, s, p):
            @pl.when(p >= 1)
            def _():
                pltpu.make_async_copy(
                    sbufs[s], out_hbm.at[pl.ds(base, CH), 0],
                    wsems[s]).wait()
            iv = idx_v[pl.ds(chunk * CH, CH)] + row_off
            pltpu.async_copy(rep_hbm.at[iv], sbufs[s], gsems[s])

        def finish_stream(chunk, s):
            pltpu.make_async_copy(
                rep_hbm.at[pl.ds(0, CH)], sbufs[s], gsems[s]).wait()
            pltpu.async_copy(
                sbufs[s], out_hbm.at[pl.ds(base + chunk * CH, CH), 0],
                wsems[s])

        def group_body(p, carry):
            start_stream(3 * p, 0, p)
            start_stream(3 * p + 1, 1, p)

            @pl.when(p >= 1)
            def _():
                pltpu.make_async_copy(
                    vbuf, out_hbm.at[pl.ds(base, CH)], wsems[2]).wait()
            rows = idx_v[pl.ds((3 * p + 2) * CH, CH)]

            @plsc.parallel_loop(0, CH, 1, unroll=2)
            def _(i):
                rsplat = rows.at[jnp.full((16,), 0, jnp.int32) + i].get(
                    mode="promise_in_bounds")
                rl = rsplat * D + lanes
                for k in range(D // 16):
                    val = plsc.load_gather(table_v, [rl + (k * 16)])
                    vbuf[i, 0, pl.ds(k * 16, 16)] = val

            pltpu.async_copy(
                vbuf, out_hbm.at[pl.ds(base + (3 * p + 2) * CH, CH)],
                wsems[2])
            finish_stream(3 * p, 0)
            finish_stream(3 * p + 1, 1)
            return carry

        n_full = n_chunks // 3
        lax.fori_loop(0, n_full, group_body, 0, unroll=False)
        for t, s in ((n_chunks - 2, 0), (n_chunks - 1, 1)):
            pltpu.make_async_copy(
                sbufs[s], out_hbm.at[pl.ds(base, CH), 0], wsems[s]).wait()
            iv = idx_v[pl.ds(t * CH, CH)] + row_off
            pltpu.async_copy(rep_hbm.at[iv], sbufs[s], gsems[s])
        for t, s in ((n_chunks - 2, 0), (n_chunks - 1, 1)):
            finish_stream(t, s)
        for s in range(2):
            pltpu.make_async_copy(
                sbufs[s], out_hbm.at[pl.ds(base, CH), 0], wsems[s]).wait()
        pltpu.make_async_copy(
            vbuf, out_hbm.at[pl.ds(base, CH)], wsems[2]).wait()

    return lookup_kernel


def kernel(level_idx, prefix_emb):
    B = level_idx.shape[0]
    V, D = prefix_emb.shape
    rep = jnp.broadcast_to(prefix_emb[None], (_NW, V, D))
    return _build(B, D, V)(
        level_idx, rep.reshape(_NW * V, D), rep.reshape(_NW * V * D))


# speedup vs baseline: 2.1084x; 1.0059x over previous
import functools

import jax
import jax.numpy as jnp
from jax import lax
from jax.experimental import pallas as pl
from jax.experimental.pallas import tpu as pltpu
from jax.experimental.pallas import tpu_sc as plsc

N_LEVELS = 8
D_MODEL = 2048

try:
    _info = plsc.get_sparse_core_info()
    _NC, _NS = _info.num_cores, _info.num_subcores
except Exception:
    _NC, _NS = 2, 16
_NW = _NC * _NS


@functools.lru_cache(maxsize=None)
def _build(B: int, D: int, V: int):
    b_per_w = B // _NW
    CH = 16
    n_chunks = b_per_w // CH
    mesh = plsc.VectorSubcoreMesh(core_axis_name="c", subcore_axis_name="s")

    @functools.partial(
        pl.kernel,
        mesh=mesh,
        compiler_params=pltpu.CompilerParams(needs_layout_passes=False),
        out_type=jax.ShapeDtypeStruct((B, 1, D), jnp.float32),
        scratch_types=[
            pltpu.VMEM((b_per_w,), jnp.int32),
            pltpu.VMEM((V * D,), jnp.float32),
            [pltpu.VMEM((CH, D), jnp.float32) for _ in range(2)],
            pltpu.VMEM((CH, 1, D), jnp.float32),
            [pltpu.SemaphoreType.DMA for _ in range(2)],
            [pltpu.SemaphoreType.DMA for _ in range(3)],
        ],
    )
    def lookup_kernel(idx_hbm, rep_hbm, repf_hbm, out_hbm,
                      idx_v, table_v, sbufs, vbuf, gsems, wsems):
        wid = lax.axis_index("s") * _NC + lax.axis_index("c")
        base = wid * b_per_w
        pltpu.sync_copy(idx_hbm.at[pl.ds(base, b_per_w)], idx_v)
        pltpu.sync_copy(repf_hbm.at[pl.ds(wid * V * D, V * D)], table_v)
        lanes = lax.iota(jnp.int32, 16)
        row_off = wid * V

        def start_stream(chunk, s, p):
            @pl.when(p >= 1)
            def _():
                pltpu.make_async_copy(
                    sbufs[s], out_hbm.at[pl.ds(base, CH), 0],
                    wsems[s]).wait()
            iv = idx_v[pl.ds(chunk * CH, CH)] + row_off
            pltpu.async_copy(rep_hbm.at[iv], sbufs[s], gsems[s])

        def finish_stream(chunk, s):
            pltpu.make_async_copy(
                rep_hbm.at[pl.ds(0, CH)], sbufs[s], gsems[s]).wait()
            pltpu.async_copy(
                sbufs[s], out_hbm.at[pl.ds(base + chunk * CH, CH), 0],
                wsems[s])

        def group_body(p, carry):
            start_stream(3 * p, 0, p)
            start_stream(3 * p + 1, 1, p)

            @pl.when(p >= 1)
            def _():
                pltpu.make_async_copy(
                    vbuf, out_hbm.at[pl.ds(base, CH)], wsems[2]).wait()
            rows = idx_v[pl.ds((3 * p + 2) * CH, CH)]

            @plsc.parallel_loop(0, CH, 1, unroll=2)
            def _(i):
                rsplat = rows.at[jnp.full((16,), 0, jnp.int32) + i].get(
                    mode="promise_in_bounds")
                rl = rsplat * D + lanes
                for k in range(D // 16):
                    val = plsc.load_gather(table_v, [rl + (k * 16)])
                    vbuf[i, 0, pl.ds(k * 16, 16)] = val

            pltpu.async_copy(
                vbuf, out_hbm.at[pl.ds(base + (3 * p + 2) * CH, CH)],
                wsems[2])
            finish_stream(3 * p, 0)
            finish_stream(3 * p + 1, 1)
            return carry

        n_full = n_chunks // 3
        lax.fori_loop(0, n_full, group_body, 0, unroll=False)
        for t, s in ((n_chunks - 2, 0), (n_chunks - 1, 1)):
            pltpu.make_async_copy(
                sbufs[s], out_hbm.at[pl.ds(base, CH), 0], wsems[s]).wait()
            iv = idx_v[pl.ds(t * CH, CH)] + row_off
            pltpu.async_copy(rep_hbm.at[iv], sbufs[s], gsems[s])
        for t, s in ((n_chunks - 2, 0), (n_chunks - 1, 1)):
            finish_stream(t, s)
        for s in range(2):
            pltpu.make_async_copy(
                sbufs[s], out_hbm.at[pl.ds(base, CH), 0], wsems[s]).wait()
        pltpu.make_async_copy(
            vbuf, out_hbm.at[pl.ds(base, CH)], wsems[2]).wait()

    return lookup_kernel


def kernel(level_idx, prefix_emb):
    B = level_idx.shape[0]
    V, D = prefix_emb.shape
    rep = jnp.broadcast_to(prefix_emb[None], (_NW, V, D))
    return _build(B, D, V)(
        level_idx, rep.reshape(_NW * V, D), rep.reshape(_NW * V * D))
